# baseline (speedup 1.0000x reference)
import functools

import jax
import jax.numpy as jnp
from jax.experimental import pallas as pl
from jax.experimental.pallas import tpu as pltpu


def _fused_kernel(x_ref, keep_ref, hef_ref, w1_ref, b1_ref, w2_ref, b2_ref,
                  out_ref, acc_ref, *, tile_n):
    ni = pl.program_id(1)
    nn = pl.num_programs(1)

    @pl.when(ni == 0)
    def _init():
        acc_ref[...] = jnp.zeros_like(acc_ref)

    keep = keep_ref[0, :, pl.ds(ni * tile_n, tile_n)]
    x = x_ref[...]
    acc_ref[...] += jnp.sum(x * keep[:, :, None], axis=1)

    @pl.when(ni == nn - 1)
    def _epilogue():
        pooled = acc_ref[...]
        keep_all = keep_ref[0]
        den = jnp.maximum(jnp.sum(keep_all, axis=1, keepdims=True), 1.0)
        mean = pooled / den
        hef = hef_ref[...]
        sf = jnp.sum(jnp.where(hef != 0.0, 0.0, mean), axis=1, keepdims=True)
        sf = jnp.where(sf == 0.0, 1.0, sf)
        scale = 10000.0 / (den * sf)
        xl = jnp.log1p(pooled * scale)
        h = jnp.maximum(
            jnp.dot(xl, w1_ref[...], preferred_element_type=jnp.float32)
            + b1_ref[...], 0.0)
        out_ref[...] = (
            jnp.dot(h, w2_ref[...], preferred_element_type=jnp.float32)
            + b2_ref[...]).astype(out_ref.dtype)[None]


def kernel(x, x_mask, he_mask, w1, b1, w2, b2):
    B, N, D = x.shape
    M = w1.shape[1]
    f32 = jnp.float32
    x = x.astype(f32)

    keep = (~x_mask).astype(f32)
    hef = he_mask.astype(f32)[None, :]
    b1r = b1.astype(f32)[None, :]
    b2r = b2.astype(f32)[None, :]
    w1 = w1.astype(f32)
    w2 = w2.astype(f32)

    bp = 2 if B % 2 == 0 else 1
    bh = B // bp

    tile_n = N
    while 2 * bh * tile_n * D * 4 > 34 * 1024 * 1024 and tile_n % 2 == 0:
        tile_n //= 2
    nn = N // tile_n
    assert nn * tile_n == N, "N must be divisible by the chosen cell tile"

    keep3 = keep.reshape(bp, bh, N)

    out = pl.pallas_call(
        functools.partial(_fused_kernel, tile_n=tile_n),
        out_shape=jax.ShapeDtypeStruct((bp, bh, M), f32),
        grid=(bp, nn),
        in_specs=[
            pl.BlockSpec((bh, tile_n, D), lambda bi, ni: (bi, ni, 0)),
            pl.BlockSpec((1, bh, N), lambda bi, ni: (bi, 0, 0)),
            pl.BlockSpec((1, D), lambda bi, ni: (0, 0)),
            pl.BlockSpec((D, M), lambda bi, ni: (0, 0)),
            pl.BlockSpec((1, M), lambda bi, ni: (0, 0)),
            pl.BlockSpec((M, M), lambda bi, ni: (0, 0)),
            pl.BlockSpec((1, M), lambda bi, ni: (0, 0)),
        ],
        out_specs=pl.BlockSpec((1, bh, M), lambda bi, ni: (bi, 0, 0)),
        scratch_shapes=[pltpu.VMEM((bh, D), f32)],
        compiler_params=pltpu.CompilerParams(
            dimension_semantics=("parallel", "arbitrary"),
            vmem_limit_bytes=56 * 1024 * 1024,
        ),
    )(x, keep3, hef, w1, b1r, w2, b2r)
    return out.reshape(B, M)


# speedup vs baseline: 1.0079x; 1.0079x over previous
import functools

import jax
import jax.numpy as jnp
from jax.experimental import pallas as pl
from jax.experimental.pallas import tpu as pltpu


def _fused_kernel(x_ref, keep_ref, hef_ref, w1_ref, b1_ref, w2_ref, b2_ref,
                  out_ref, acc_ref, *, tile_n):
    ni = pl.program_id(1)
    nn = pl.num_programs(1)

    @pl.when(ni == 0)
    def _init():
        acc_ref[...] = jnp.zeros_like(acc_ref)

    keep = keep_ref[0, :, pl.ds(ni * tile_n, tile_n)]
    x = x_ref[...]
    acc_ref[...] += jnp.sum(x * keep[:, :, None], axis=1)

    @pl.when(ni == nn - 1)
    def _epilogue():
        pooled = acc_ref[...]
        keep_all = keep_ref[0]
        den = jnp.maximum(jnp.sum(keep_all, axis=1, keepdims=True), 1.0)
        mean = pooled / den
        hef = hef_ref[...]
        sf = jnp.sum(jnp.where(hef != 0.0, 0.0, mean), axis=1, keepdims=True)
        sf = jnp.where(sf == 0.0, 1.0, sf)
        scale = 10000.0 / (den * sf)
        xl = jnp.log1p(pooled * scale)
        h = jnp.maximum(
            jnp.dot(xl, w1_ref[...], preferred_element_type=jnp.float32)
            + b1_ref[...], 0.0)
        out_ref[...] = (
            jnp.dot(h, w2_ref[...], preferred_element_type=jnp.float32)
            + b2_ref[...]).astype(out_ref.dtype)[None]


def kernel(x, x_mask, he_mask, w1, b1, w2, b2):
    B, N, D = x.shape
    M = w1.shape[1]
    f32 = jnp.float32
    x = x.astype(f32)

    keep = (~x_mask).astype(f32)
    hef = he_mask.astype(f32)[None, :]
    b1r = b1.astype(f32)[None, :]
    b2r = b2.astype(f32)[None, :]
    w1 = w1.astype(f32)
    w2 = w2.astype(f32)

    bp = 2 if B % 2 == 0 else 1
    bh = B // bp

    tile_n = N
    while 2 * bh * tile_n * D * 4 > 18 * 1024 * 1024 and tile_n % 2 == 0:
        tile_n //= 2
    nn = N // tile_n
    assert nn * tile_n == N, "N must be divisible by the chosen cell tile"

    keep3 = keep.reshape(bp, bh, N)

    out = pl.pallas_call(
        functools.partial(_fused_kernel, tile_n=tile_n),
        out_shape=jax.ShapeDtypeStruct((bp, bh, M), f32),
        grid=(bp, nn),
        in_specs=[
            pl.BlockSpec((bh, tile_n, D), lambda bi, ni: (bi, ni, 0)),
            pl.BlockSpec((1, bh, N), lambda bi, ni: (bi, 0, 0)),
            pl.BlockSpec((1, D), lambda bi, ni: (0, 0)),
            pl.BlockSpec((D, M), lambda bi, ni: (0, 0)),
            pl.BlockSpec((1, M), lambda bi, ni: (0, 0)),
            pl.BlockSpec((M, M), lambda bi, ni: (0, 0)),
            pl.BlockSpec((1, M), lambda bi, ni: (0, 0)),
        ],
        out_specs=pl.BlockSpec((1, bh, M), lambda bi, ni: (bi, 0, 0)),
        scratch_shapes=[pltpu.VMEM((bh, D), f32)],
        compiler_params=pltpu.CompilerParams(
            dimension_semantics=("parallel", "arbitrary"),
            vmem_limit_bytes=56 * 1024 * 1024,
        ),
    )(x, keep3, hef, w1, b1r, w2, b2r)
    return out.reshape(B, M)
